# baseline (speedup 1.0000x reference)
import functools

import jax
import jax.numpy as jnp
from jax import lax
from jax.experimental import pallas as pl
from jax.experimental.pallas import tpu as pltpu
from jax.experimental.pallas import tpu_sc as plsc

NUM_AXIOMS = 4096
D_AXIOM = 449
D_PAD = 464
D_TILE = 456
BATCH = 16384

_NUM_CORES = 2
_NUM_SUBCORES = 16
_NW = _NUM_CORES * _NUM_SUBCORES
_B_PER_W = BATCH // _NW
_CHUNK = 64
_NCHUNK = _B_PER_W // _CHUNK
_ROWS_PER_SUB = NUM_AXIOMS // _NUM_SUBCORES
_NSTAGE = _ROWS_PER_SUB // _CHUNK

_mesh = plsc.VectorSubcoreMesh(core_axis_name="c", subcore_axis_name="s")


@functools.partial(
    pl.kernel,
    mesh=_mesh,
    out_type=jax.ShapeDtypeStruct((BATCH, D_AXIOM), jnp.float32),
    compiler_params=pltpu.CompilerParams(
        use_tc_tiling_on_sc=False, needs_layout_passes=False
    ),
    scratch_types=[
        pltpu.VMEM((_NCHUNK, _CHUNK), jnp.int32),
        pltpu.VMEM((_CHUNK, D_PAD), jnp.float32),
        pltpu.VMEM((_CHUNK, D_PAD), jnp.float32),
        pltpu.VMEM((_CHUNK, D_AXIOM), jnp.float32),
        pltpu.VMEM((_CHUNK, D_AXIOM), jnp.float32),
        pltpu.HBM((NUM_AXIOMS, D_PAD), jnp.float32),
        pltpu.SemaphoreType.DMA,
        pltpu.SemaphoreType.DMA,
        pltpu.SemaphoreType.DMA,
        pltpu.SemaphoreType.DMA,
    ],
)
def _gather_kernel(idx_hbm, table_hbm, out_hbm,
                   idx_v, rows0, rows1, stage0, stage1, padded_hbm,
                   sg0, sg1, sw0, sw1):
    sid = lax.axis_index("s")
    wid = sid * _NUM_CORES + lax.axis_index("c")
    base = wid * _B_PER_W
    rows = (rows0, rows1)
    stages = (stage0, stage1)
    sgs = (sg0, sg1)
    sws = (sw0, sw1)

    pltpu.sync_copy(idx_hbm.at[wid], idx_v)

    row0 = sid * _ROWS_PER_SUB
    scp = [
        pltpu.async_copy(table_hbm.at[pl.ds(row0, _CHUNK)], stage0, sg0),
        None,
    ]
    wpc = [None, None]
    for q in range(_NSTAGE):
        cur = q % 2
        nxt = (q + 1) % 2
        if q + 1 < _NSTAGE:
            scp[nxt] = pltpu.async_copy(
                table_hbm.at[pl.ds(row0 + (q + 1) * _CHUNK, _CHUNK)],
                stages[nxt], sgs[nxt],
            )
        scp[cur].wait()
        if wpc[cur] is not None:
            wpc[cur].wait()
        wpc[cur] = pltpu.async_copy(
            stages[cur].at[:, pl.ds(0, D_TILE)],
            padded_hbm.at[pl.ds(row0 + q * _CHUNK, _CHUNK), pl.ds(0, D_TILE)],
            sws[cur],
        )
    wpc[0].wait()
    wpc[1].wait()
    plsc.subcore_barrier()

    gcp = [pltpu.async_copy(padded_hbm.at[idx_v.at[0]], rows0, sg0), None]
    wcp = [None, None]
    for j in range(_NCHUNK):
        cur = j % 2
        nxt = (j + 1) % 2
        if j + 1 < _NCHUNK:
            gcp[nxt] = pltpu.async_copy(
                padded_hbm.at[idx_v.at[j + 1]], rows[nxt], sgs[nxt]
            )
        gcp[cur].wait()
        if wcp[cur] is not None:
            wcp[cur].wait()
        wcp[cur] = pltpu.async_copy(
            rows[cur].at[:, pl.ds(0, D_TILE)],
            out_hbm.at[pl.ds(base + j * _CHUNK, _CHUNK), pl.ds(0, D_TILE)],
            sws[cur],
        )
    wcp[0].wait()
    wcp[1].wait()


def kernel(indices, table):
    idx = indices.astype(jnp.int32).reshape(_NW, _NCHUNK, _CHUNK)
    return _gather_kernel(idx, table)


# speedup vs baseline: 1.0873x; 1.0873x over previous
import functools

import jax
import jax.numpy as jnp
from jax import lax
from jax.experimental import pallas as pl
from jax.experimental.pallas import tpu as pltpu
from jax.experimental.pallas import tpu_sc as plsc

NUM_AXIOMS = 4096
D_AXIOM = 449
D_PAD = 464
D_TILE = 456
BATCH = 16384

_NUM_CORES = 2
_NUM_SUBCORES = 16
_NW = _NUM_CORES * _NUM_SUBCORES
_B_PER_W = BATCH // _NW
_CHUNK = 128
_NCHUNK = _B_PER_W // _CHUNK

_mesh = plsc.VectorSubcoreMesh(core_axis_name="c", subcore_axis_name="s")


@functools.partial(
    pl.kernel,
    mesh=_mesh,
    out_type=jax.ShapeDtypeStruct((BATCH, D_AXIOM), jnp.float32),
    compiler_params=pltpu.CompilerParams(
        use_tc_tiling_on_sc=False, needs_layout_passes=False
    ),
    scratch_types=[
        pltpu.VMEM((_NCHUNK, _CHUNK), jnp.int32),
        pltpu.VMEM((_CHUNK, D_PAD), jnp.float32),
        pltpu.VMEM((_CHUNK, D_PAD), jnp.float32),
        pltpu.SemaphoreType.DMA,
        pltpu.SemaphoreType.DMA,
        pltpu.SemaphoreType.DMA,
        pltpu.SemaphoreType.DMA,
    ],
)
def _gather_kernel(idx_hbm, table_hbm, out_hbm,
                   idx_v, rows0, rows1, sg0, sg1, sw0, sw1):
    wid = lax.axis_index("s") * _NUM_CORES + lax.axis_index("c")
    base = wid * _B_PER_W
    pltpu.sync_copy(idx_hbm.at[wid], idx_v)
    rows = (rows0, rows1)
    sgs = (sg0, sg1)
    sws = (sw0, sw1)
    gcp = [pltpu.async_copy(table_hbm.at[idx_v.at[0]], rows0, sg0), None]
    wcp = [None, None]
    for j in range(_NCHUNK):
        cur = j % 2
        nxt = (j + 1) % 2
        if wcp[nxt] is not None:
            wcp[nxt].wait()
        if j + 1 < _NCHUNK:
            gcp[nxt] = pltpu.async_copy(
                table_hbm.at[idx_v.at[j + 1]], rows[nxt], sgs[nxt]
            )
        gcp[cur].wait()
        wcp[cur] = pltpu.async_copy(
            rows[cur].at[:, pl.ds(0, D_TILE)],
            out_hbm.at[pl.ds(base + j * _CHUNK, _CHUNK), pl.ds(0, D_TILE)],
            sws[cur],
        )
    wcp[(_NCHUNK - 1) % 2].wait()


def kernel(indices, table):
    idx = indices.astype(jnp.int32).reshape(_NW, _NCHUNK, _CHUNK)
    table_pad = jnp.pad(table, ((0, 0), (0, D_PAD - D_AXIOM)))
    return _gather_kernel(idx, table_pad)
